# Initial kernel scaffold; baseline (speedup 1.0000x reference)
import jax
import jax.numpy as jnp
from jax.experimental import pallas as pl


def kernel(data, params):
    raise NotImplementedError("write your pallas kernel here")


# speedup vs baseline: 74.1120x; 74.1120x over previous
import jax
import jax.numpy as jnp
from jax.experimental import pallas as pl

NN = 128
C = 64
T = 64
B = 16
TOPK = 20
TACT = 16
NROWS = 136
BG_W = 240.0
BN_CNT = 2048.0 * T
_PREC = jax.lax.Precision.HIGHEST


def _mm(a, b):
    sh = a.shape
    a2 = a.reshape(-1, sh[-1]).astype(jnp.bfloat16)
    r = jax.lax.dot_general(a2, b.astype(jnp.bfloat16), (((1,), (0,)), ((), ())),
                            preferred_element_type=jnp.float32)
    return r.reshape(sh[:-1] + (b.shape[-1],))


def _tshift(x, s):
    if s == 0:
        return x
    r, t, c = x.shape
    z = jnp.zeros((r, abs(s), c), x.dtype)
    if s > 0:
        return jnp.concatenate([x[:, s:, :], z], axis=1)
    return jnp.concatenate([z, x[:, :t + s, :]], axis=1)


def _body(data_ref, emb_ref, attij_ref, linv_ref, vecs_ref, w3_ref, w5_ref,
          dwv_ref, pw_ref, fw_ref, outwT_ref, outb_ref, out_ref):
    f32 = jnp.float32
    emb = emb_ref[...]
    linv = linv_ref[...]

    ebf = emb.astype(jnp.bfloat16)
    g = jax.lax.dot_general(ebf, ebf.T, (((1,), (0,)), ((), ())),
                            preferred_element_type=jnp.float32)
    nrm = jnp.sqrt(jnp.sum(emb * emb, axis=1, keepdims=True))
    cos = g / (nrm * nrm.T)
    jidx = jax.lax.broadcasted_iota(jnp.int32, (NN, NN), 1)
    mask = jnp.zeros((NN, NN), f32)
    cm = cos
    for _ in range(TOPK):
        rmax = jnp.max(cm, axis=1, keepdims=True)
        cand = jnp.where(cm == rmax, jidx, NN)
        jmin = jnp.min(cand, axis=1, keepdims=True)
        one = jidx == jmin
        mask = jnp.where(one, 1.0, mask)
        cm = jnp.where(one, -jnp.inf, cm)

    att_i = attij_ref[0:1, :]
    att_j = attij_ref[1:2, :]
    a_i = jnp.sum(linv * att_i[:, :C])
    a_j = jnp.sum(linv * att_j[:, :C])
    e_i = jnp.sum(emb * att_i[:, C:], axis=1, keepdims=True)
    e_j = jnp.sum(emb * att_j[:, C:], axis=1, keepdims=True)

    xa = data_ref[0, :, 0:TACT]
    xt = xa.T
    alpha = (xt[:, :, None] * a_i + e_i.reshape(1, NN, 1)
             + xt[:, None, :] * a_j + e_j.reshape(1, 1, NN))
    alpha = jnp.where(alpha >= 0, alpha, 0.2 * alpha)
    am = jnp.where(mask[None, :, :] > 0, alpha, -jnp.inf)
    amax = jnp.max(am, axis=2, keepdims=True)
    ex = jnp.exp(am - amax)
    denom = jnp.sum(ex, axis=2, keepdims=True) + 1e-16
    att = ex / denom
    s = jnp.sum(att * xt[:, None, :], axis=2)

    gnn_bias = vecs_ref[22:23, :].reshape(1, 1, C)
    a_act = s.T[:, :, None] * linv.reshape(1, 1, C)
    a128 = jnp.concatenate(
        [a_act, jnp.zeros((NN, T - TACT, C), f32)], axis=1)
    x_all = jnp.concatenate(
        [a128, jnp.zeros((NROWS - NN, T, C), f32)], axis=0) + gnn_bias

    ridx = jax.lax.broadcasted_iota(jnp.int32, (NROWS, 1, 1), 0)
    wr = jnp.where(ridx < NN, 1.0, BG_W)

    def bn(x, gi, bi):
        gv = vecs_ref[gi:gi + 1, :].reshape(1, 1, C)
        bv = vecs_ref[bi:bi + 1, :].reshape(1, 1, C)
        m = jnp.sum(x * wr, axis=(0, 1), keepdims=True) / BN_CNT
        d = x - m
        v = jnp.sum(d * d * wr, axis=(0, 1), keepdims=True) / BN_CNT
        return d / jnp.sqrt(v + 1e-5) * gv + bv

    def conv(x, wref, blk, k, pad, bias_row):
        acc = None
        for dk in range(k):
            t = _mm(_tshift(x, dk - pad), wref[blk, dk])
            acc = t if acc is None else acc + t
        return acc + vecs_ref[bias_row:bias_row + 1, :].reshape(1, 1, C)

    def ms_block(x, blk):
        v0 = blk * 11
        res = None
        for dk in range(3):
            t = _tshift(x, dk - 1) * dwv_ref[blk * 3 + dk, :].reshape(1, 1, C)
            res = t if res is None else res + t
        res = res + vecs_ref[v0 + 6:v0 + 7, :].reshape(1, 1, C)
        res = _mm(res, pw_ref[blk]) + vecs_ref[v0 + 7:v0 + 8, :].reshape(1, 1, C)
        res = bn(res, v0 + 8, v0 + 9)
        b1 = jax.nn.relu(bn(conv(x, w3_ref, blk, 3, 1, v0 + 0), v0 + 1, v0 + 2))
        b2 = jax.nn.relu(bn(conv(x, w5_ref, blk, 5, 2, v0 + 3), v0 + 4, v0 + 5))
        fused = _mm(jnp.concatenate([b1, b2], axis=2), fw_ref[blk])
        fused = fused + vecs_ref[v0 + 10:v0 + 11, :].reshape(1, 1, C)
        return jax.nn.relu(fused + res)

    h = ms_block(ms_block(x_all, 0), 1)
    hm = jnp.sum(h, axis=1) / float(T)

    y0 = hm[:NN, :] * emb
    ybg = hm[NN:NN + 1, :] * emb
    m = jnp.sum(y0 + 15.0 * ybg, axis=0, keepdims=True) / 2048.0
    d0 = y0 - m
    dbg = ybg - m
    v = (jnp.sum(d0 * d0, axis=0, keepdims=True)
         + 15.0 * jnp.sum(dbg * dbg, axis=0, keepdims=True)) / 2048.0
    bno_g = vecs_ref[23:24, :]
    bno_b = vecs_ref[24:25, :]
    z0 = jax.nn.relu(d0 / jnp.sqrt(v + 1e-5) * bno_g + bno_b)
    zbg = jax.nn.relu(dbg / jnp.sqrt(v + 1e-5) * bno_g + bno_b)
    row0 = (_mm(z0, outwT_ref[...]) + outb_ref[0, 0]).T
    rbg = (_mm(zbg, outwT_ref[...]) + outb_ref[0, 0]).T
    out_ref[...] = jnp.concatenate(
        [row0, jnp.broadcast_to(rbg, (B - 1, NN))], axis=0)


def kernel(data, params):
    p = params
    f32 = jnp.float32

    vec_rows = []
    for pfx in ('tcn1_', 'tcn2_'):
        vec_rows += [p[pfx + 'c3_b'], p[pfx + 'bn1_g'], p[pfx + 'bn1_b'],
                     p[pfx + 'c5_b'], p[pfx + 'bn2_g'], p[pfx + 'bn2_b'],
                     p[pfx + 'dw_b'], p[pfx + 'pw_b'], p[pfx + 'bn3_g'],
                     p[pfx + 'bn3_b'], p[pfx + 'fus_b']]
    vec_rows += [p['gnn_bias'], p['bno_g'], p['bno_b']]
    vecs = jnp.stack(vec_rows).astype(f32)

    w3 = jnp.stack([p['tcn1_c3_W'].transpose(2, 1, 0),
                    p['tcn2_c3_W'].transpose(2, 1, 0)])
    w5 = jnp.stack([p['tcn1_c5_W'].transpose(2, 1, 0),
                    p['tcn2_c5_W'].transpose(2, 1, 0)])
    dwv = jnp.concatenate([p['tcn1_dw_W'][:, 0, :].T,
                           p['tcn2_dw_W'][:, 0, :].T])
    pw = jnp.stack([p['tcn1_pw_W'][:, :, 0].T,
                    p['tcn2_pw_W'][:, :, 0].T])
    fw = jnp.stack([p['tcn1_fus_W'][:, :, 0].T,
                    p['tcn2_fus_W'][:, :, 0].T])
    attij = jnp.stack([p['att_i'], p['att_j']])
    linv = p['lin_W'][:, 0].reshape(1, C)
    outwT = p['out_W'].T
    outb = p['out_b'].reshape(1, 1)

    return pl.pallas_call(
        _body,
        out_shape=jax.ShapeDtypeStruct((B, NN), f32),
    )(data.astype(f32), p['emb'].astype(f32), attij, linv, vecs,
      w3, w5, dwv, pw, fw, outwT, outb)
